# Initial kernel scaffold; baseline (speedup 1.0000x reference)
import jax
import jax.numpy as jnp
from jax.experimental import pallas as pl


def kernel(e_conv1_w, e_conv1_b, e_conv1_1_w, e_conv1_1_b, e_conv2_w, e_conv2_b, e_conv2_1_w, e_conv2_1_b, e_conv3_w, e_conv3_b, e_conv3_1_w, e_conv3_1_b, e_conv4_w, e_conv4_b, e_conv4_1_w, e_conv4_1_b, e_fc1_w, e_fc1_b, e_fc2_w, e_fc2_b, d_fc1_w, d_fc1_b, d_fc2_w, d_fc2_b, d_fc3_w, d_fc3_b, d_ct1_w, d_ct1_b, d_ct2_w, d_ct2_b, d_ct3_w, d_ct3_b, d_ct4_w, d_ct4_b, observation, eps):
    raise NotImplementedError("write your pallas kernel here")


# speedup vs baseline: 1.0280x; 1.0280x over previous
import functools
import math

import jax
import jax.numpy as jnp
from jax.experimental import pallas as pl
from jax.experimental.pallas import tpu as pltpu

_VMEM_LIMIT = 100 * 1024 * 1024
_TILE_M = 8192


def _gemm_body(*refs, act, nbias):
    if nbias:
        x_ref, w_ref, b_ref, o_ref = refs
    else:
        x_ref, w_ref, o_ref = refs
    acc = jnp.dot(x_ref[...], w_ref[...], preferred_element_type=jnp.float32)
    if nbias:
        acc = acc + b_ref[...]
    if act == "relu":
        acc = jnp.maximum(acc, 0.0)
    elif act == "sigmoid":
        acc = jax.nn.sigmoid(acc)
    o_ref[...] = acc.astype(o_ref.dtype)


def _gemm(x, w, b=None, act="none", out_dtype=jnp.bfloat16, tile_m=_TILE_M):
    M, K = x.shape
    N = w.shape[1]
    tm = min(M, tile_m)
    grid = (pl.cdiv(M, tm),)
    in_specs = [pl.BlockSpec((tm, K), lambda i: (i, 0)),
                pl.BlockSpec((K, N), lambda i: (0, 0))]
    args = [x, w]
    if b is not None:
        in_specs.append(pl.BlockSpec((1, N), lambda i: (0, 0)))
        args.append(b.reshape(1, N).astype(jnp.float32))
    return pl.pallas_call(
        functools.partial(_gemm_body, act=act, nbias=b is not None),
        out_shape=jax.ShapeDtypeStruct((M, N), out_dtype),
        grid=grid,
        in_specs=in_specs,
        out_specs=pl.BlockSpec((tm, N), lambda i: (i, 0)),
        compiler_params=pltpu.CompilerParams(
            dimension_semantics=("parallel",),
            vmem_limit_bytes=_VMEM_LIMIT),
    )(*args)


def _patches(x, kh, kw, stride, padding):
    b, h, w, c = x.shape
    if padding:
        x = jnp.pad(x, ((0, 0), (padding, padding), (padding, padding), (0, 0)))
    hp, wp = h + 2 * padding, w + 2 * padding
    oh = (hp - kh) // stride + 1
    ow = (wp - kw) // stride + 1
    taps = []
    for i in range(kh):
        for j in range(kw):
            taps.append(jax.lax.slice(
                x, (0, i, j, 0),
                (b, i + stride * (oh - 1) + 1, j + stride * (ow - 1) + 1, c),
                (1, stride, stride, 1)))
    cols = jnp.stack(taps, axis=3).reshape(b * oh * ow, kh * kw * c)
    return cols, oh, ow


def _conv(x, w, bias, stride, padding, act):
    cout, cin, kh, kw = w.shape
    cols, oh, ow = _patches(x, kh, kw, stride, padding)
    w2 = jnp.transpose(w, (2, 3, 1, 0)).reshape(kh * kw * cin, cout)
    y = _gemm(cols, w2.astype(jnp.bfloat16), bias, act)
    return y.reshape(x.shape[0], oh, ow, cout)


def _deconv(x, w, bias, stride, act, out_dtype=jnp.bfloat16):
    cin, cout, kh, kw = w.shape
    b, h, wd, _ = x.shape
    s = stride
    oh, ow = (h - 1) * s + kh, (wd - 1) * s + kw
    w2 = jnp.transpose(w, (0, 2, 3, 1)).reshape(cin, kh * kw * cout)
    y = _gemm(x.reshape(b * h * wd, cin), w2.astype(jnp.bfloat16), None, "none")
    y = y.reshape(b, h, wd, kh, kw, cout)

    na, nb = -(-kh // s), -(-kw // s)
    hs, ws = h + na - 1, wd + nb - 1
    classes = []
    for pr in range(s):
        row = []
        for pc in range(s):
            acc = jnp.zeros((b, hs, ws, cout), jnp.float32)
            for a in range(na):
                ki = pr + s * a
                if ki >= kh:
                    continue
                for bb in range(nb):
                    kj = pc + s * bb
                    if kj >= kw:
                        continue
                    tap = y[:, :, :, ki, kj, :].astype(jnp.float32)
                    acc = acc + jnp.pad(
                        tap, ((0, 0), (a, hs - h - a), (bb, ws - wd - bb),
                              (0, 0)))
            row.append(acc)
        classes.append(jnp.stack(row, axis=0))
    full = jnp.stack(classes, axis=0)
    full = jnp.transpose(full, (2, 3, 0, 4, 1, 5))
    out = full.reshape(b, hs * s, ws * s, cout)[:, :oh, :ow, :]
    out = out + bias.astype(jnp.float32)
    if act == "relu":
        out = jnp.maximum(out, 0.0)
    elif act == "sigmoid":
        out = jax.nn.sigmoid(out)
    return out.astype(out_dtype)


def _enc_head_body(x_ref, w1_ref, b1_ref, w2_ref, b2_ref, o_ref):
    h = jnp.dot(x_ref[...], w1_ref[...], preferred_element_type=jnp.float32)
    h = jnp.maximum(h + b1_ref[...], 0.0)
    o = jnp.dot(h, w2_ref[...], preferred_element_type=jnp.float32)
    o_ref[...] = o + b2_ref[...]


def _enc_head(x, w1, b1, w2, b2):
    m = x.shape[0]
    n1, n2 = w1.shape[1], w2.shape[1]
    tm = m // 2
    operands = [x, w1.astype(jnp.float32), b1.astype(jnp.float32).reshape(1, n1),
                w2.astype(jnp.float32), b2.astype(jnp.float32).reshape(1, n2)]
    in_specs = [pl.BlockSpec((tm, x.shape[1]), lambda i: (i, 0))]
    in_specs += [pl.BlockSpec(op.shape, lambda i: (0, 0)) for op in operands[1:]]
    return pl.pallas_call(
        _enc_head_body,
        out_shape=jax.ShapeDtypeStruct((m, n2), jnp.float32),
        grid=(2,),
        in_specs=in_specs,
        out_specs=pl.BlockSpec((tm, n2), lambda i: (i, 0)),
        compiler_params=pltpu.CompilerParams(
            dimension_semantics=("parallel",),
            vmem_limit_bytes=_VMEM_LIMIT),
    )(*operands)


def _dec_head_body(x_ref, w1_ref, b1_ref, w2_ref, b2_ref, w3_ref, b3_ref,
                   w4_ref, b4_ref, o_ref):
    h = jnp.dot(x_ref[...], w1_ref[...], preferred_element_type=jnp.float32)
    h = jnp.maximum(h + b1_ref[...], 0.0)
    h = jnp.dot(h, w2_ref[...], preferred_element_type=jnp.float32)
    h = jnp.maximum(h + b2_ref[...], 0.0)
    h = jnp.dot(h, w3_ref[...], preferred_element_type=jnp.float32) + b3_ref[...]
    y = jnp.dot(h, w4_ref[...], preferred_element_type=jnp.float32) + b4_ref[...]
    o_ref[...] = jnp.maximum(y, 0.0).astype(o_ref.dtype)


def _dec_head(x, ws_bs):
    m = x.shape[0]
    operands = [x]
    in_specs = [pl.BlockSpec((m // 2, x.shape[1]), lambda i: (i, 0))]
    for w, b in ws_bs:
        n = w.shape[1]
        operands += [w.astype(jnp.float32), b.astype(jnp.float32).reshape(1, n)]
        in_specs += [pl.BlockSpec(w.shape, lambda i: (0, 0)),
                     pl.BlockSpec((1, n), lambda i: (0, 0))]
    n_out = ws_bs[-1][0].shape[1]
    return pl.pallas_call(
        _dec_head_body,
        out_shape=jax.ShapeDtypeStruct((m, n_out), jnp.bfloat16),
        grid=(2,),
        in_specs=in_specs,
        out_specs=pl.BlockSpec((m // 2, n_out), lambda i: (i, 0)),
        compiler_params=pltpu.CompilerParams(
            dimension_semantics=("parallel",),
            vmem_limit_bytes=_VMEM_LIMIT),
    )(*operands)


def kernel(e_conv1_w, e_conv1_b, e_conv1_1_w, e_conv1_1_b, e_conv2_w,
           e_conv2_b, e_conv2_1_w, e_conv2_1_b, e_conv3_w, e_conv3_b,
           e_conv3_1_w, e_conv3_1_b, e_conv4_w, e_conv4_b, e_conv4_1_w,
           e_conv4_1_b, e_fc1_w, e_fc1_b, e_fc2_w, e_fc2_b, d_fc1_w, d_fc1_b,
           d_fc2_w, d_fc2_b, d_fc3_w, d_fc3_b, d_ct1_w, d_ct1_b, d_ct2_w,
           d_ct2_b, d_ct3_w, d_ct3_b, d_ct4_w, d_ct4_b, observation, eps):
    hidden_size = 32
    B, T = observation.shape[0], observation.shape[1]
    bt = B * T

    x = observation.reshape(bt, 3, 64, 64)
    x = jnp.transpose(x, (0, 2, 3, 1)).astype(jnp.bfloat16)
    x = _conv(x, e_conv1_w, e_conv1_b, 2, 0, "relu")
    x = _conv(x, e_conv1_1_w, e_conv1_1_b, 1, 1, "relu")
    x = _conv(x, e_conv2_w, e_conv2_b, 2, 0, "relu")
    x = _conv(x, e_conv2_1_w, e_conv2_1_b, 1, 1, "relu")
    x = _conv(x, e_conv3_w, e_conv3_b, 2, 0, "relu")
    x = _conv(x, e_conv3_1_w, e_conv3_1_b, 1, 1, "relu")
    x = _conv(x, e_conv4_w, e_conv4_b, 2, 0, "relu")
    x = _conv(x, e_conv4_1_w, e_conv4_1_b, 1, 1, "relu")
    x = jnp.transpose(x, (0, 3, 1, 2)).reshape(bt, -1)
    hid = _enc_head(x.astype(jnp.float32), e_fc1_w.T, e_fc1_b,
                    e_fc2_w.T, e_fc2_b)

    mu2 = hid[:, :hidden_size]
    std2 = jnp.maximum(hid[:, hidden_size:], 1e-5)
    enc2 = mu2 + std2 * eps.reshape(bt, hidden_size)
    kl = -jnp.log(std2) + 0.5 * (std2 * std2 + mu2 * mu2) - 0.5
    klloss = jnp.mean(kl)

    cin, cout, kh, kw = d_ct1_w.shape
    w4 = jnp.transpose(d_ct1_w, (0, 2, 3, 1)).reshape(cin, kh * kw * cout)
    b4 = jnp.tile(d_ct1_b, kh * kw)
    y = _dec_head(enc2, [(d_fc1_w.T, d_fc1_b), (d_fc2_w.T, d_fc2_b),
                         (d_fc3_w.T, d_fc3_b), (w4, b4)])
    x = y.reshape(bt, kh, kw, cout)
    x = _deconv(x, d_ct2_w, d_ct2_b, 2, "relu")
    x = _deconv(x, d_ct3_w, d_ct3_b, 2, "relu")
    x = _deconv(x, d_ct4_w, d_ct4_b, 2, "sigmoid", jnp.float32)
    rec = jnp.transpose(x, (0, 3, 1, 2)).reshape(B, T, 3, 64, 64)

    mu = mu2.reshape(B, T, hidden_size)
    std = std2.reshape(B, T, hidden_size)
    encoding = enc2.reshape(B, T, hidden_size)
    return rec, mu, std, klloss, encoding
